# baseline (speedup 1.0000x reference)
import functools

import jax
import jax.numpy as jnp
from jax import lax
from jax.experimental import pallas as pl
from jax.experimental.pallas import tpu as pltpu
from jax.experimental.pallas import tpu_sc as plsc

B = 16384
ED = 16
NC = 2
NS = 16
NW = NC * NS
BPW = B // NW
CHUNK = 128
NCHUNK = BPW // CHUNK
HALF = 2 * CHUNK


def _gather6(idx_packed, *tables_sr):
    mesh = plsc.VectorSubcoreMesh(core_axis_name="c", subcore_axis_name="s")

    @functools.partial(
        pl.kernel,
        out_type=[jax.ShapeDtypeStruct((B, 128), jnp.float32) for _ in range(6)],
        mesh=mesh,
        scratch_types=[
            pltpu.VMEM((24, CHUNK), jnp.int32),
            pltpu.VMEM((HALF, 128), jnp.float32),
            pltpu.VMEM((HALF, 128), jnp.float32),
            pltpu.SemaphoreType.DMA,
            pltpu.SemaphoreType.DMA,
            pltpu.SemaphoreType.DMA,
        ],
    )
    def k(idx_hbm, tn, td, tt, to, ta, tk,
          o0, o1, o2, o3, o4, o5, idx_v, bufa, bufb, gsem, wsa, wsb):
        wid = lax.axis_index("s") * NC + lax.axis_index("c")
        base = wid * BPW
        tabs = [tn, td, tt, to, ta, tk]
        outs = [o0, o1, o2, o3, o4, o5]
        bufs = [bufa, bufb]
        wsems = [wsa, wsb]
        pltpu.sync_copy(idx_hbm.at[pl.ds(wid * 24, 24)], idx_v)
        pending = [None, None]
        for d in range(12):
            t, h = divmod(d, 2)
            p = d % 2
            if pending[p] is not None:
                pending[p].wait()
            g0 = pltpu.async_copy(
                tabs[t].at[idx_v.at[t * 4 + 2 * h]],
                bufs[p].at[pl.ds(0, CHUNK)], gsem)
            g1 = pltpu.async_copy(
                tabs[t].at[idx_v.at[t * 4 + 2 * h + 1]],
                bufs[p].at[pl.ds(CHUNK, CHUNK)], gsem)
            g0.wait()
            g1.wait()
            pending[p] = pltpu.async_copy(
                bufs[p], outs[t].at[pl.ds(base + h * HALF, HALF)], wsems[p])
        pending[0].wait()
        pending[1].wait()

    return k(idx_packed, *tables_sr)


def _mlp_body(r0, r1, r2, r3, r4, r5, s8, nf, w1, b1, w2, b2, out):
    raws = [r0, r1, r2, r3, r4, r5]
    es = []
    for t in range(6):
        raw = raws[t][:]
        sc = s8[:, t:t + 1]
        e = jnp.where(sc == 0, raw[:, 0:ED], 0.0)
        for kk in range(1, 8):
            e = e + jnp.where(sc == kk, raw[:, kk * ED:(kk + 1) * ED], 0.0)
        es.append(e)
    num = jnp.dot(nf[:], w1[:], preferred_element_type=jnp.float32) + b1[:]
    feats = jnp.concatenate(es + [num], axis=-1)
    acc = jnp.dot(feats, w2[:], preferred_element_type=jnp.float32) + b2[:]
    out[:] = jnp.maximum(acc, 0.0)


def _mlp(raw_list, s8, nf, w1, b1, w2, b2):
    BB = 1024
    grid = (B // BB,)
    rspec = pl.BlockSpec((BB, 128), lambda i: (i, 0))
    return pl.pallas_call(
        _mlp_body,
        grid=grid,
        in_specs=[rspec] * 6 + [
            pl.BlockSpec((BB, 8), lambda i: (i, 0)),
            pl.BlockSpec((BB, 6), lambda i: (i, 0)),
            pl.BlockSpec((6, 20), lambda i: (0, 0)),
            pl.BlockSpec((1, 20), lambda i: (0, 0)),
            pl.BlockSpec((116, 64), lambda i: (0, 0)),
            pl.BlockSpec((1, 64), lambda i: (0, 0)),
        ],
        out_specs=pl.BlockSpec((BB, 64), lambda i: (i, 0)),
        out_shape=jax.ShapeDtypeStruct((B, 64), jnp.float32),
        compiler_params=pltpu.CompilerParams(
            dimension_semantics=("parallel",),
        ),
    )(*raw_list, s8, nf, w1, b1, w2, b2)


def kernel(model_name, pretrained_dataset, model_type, model_owner,
           model_architecture, model_task, numeric_features,
           T_name, T_ds, T_type, T_owner, T_arch, T_task, W1, b1, W2, b2):
    idx = jnp.stack([
        model_name.astype(jnp.int32),
        pretrained_dataset.astype(jnp.int32),
        model_type.astype(jnp.int32),
        model_owner.astype(jnp.int32),
        model_architecture.astype(jnp.int32),
        model_task.astype(jnp.int32),
    ], axis=0)
    sup = idx >> 3
    sub = idx & 7
    sup = (sup.reshape(6, NW, NCHUNK, CHUNK)
              .transpose(1, 0, 2, 3)
              .reshape(NW * 24, CHUNK))
    s8 = jnp.concatenate(
        [sub.T, jnp.zeros((B, 2), jnp.int32)], axis=1)
    tables = [T_name, T_ds, T_type, T_owner, T_arch, T_task]
    tables_sr = [t.reshape(t.shape[0] // 8, 128) for t in tables]
    raw = _gather6(sup, *tables_sr)
    return _mlp(raw, s8, numeric_features,
                W1, b1.reshape(1, 20), W2, b2.reshape(1, 64))


# speedup vs baseline: 1.9804x; 1.9804x over previous
import functools

import jax
import jax.numpy as jnp
from jax import lax
from jax.experimental import pallas as pl
from jax.experimental.pallas import tpu as pltpu
from jax.experimental.pallas import tpu_sc as plsc

B = 16384
ED = 16
NC = 2
NS = 16
NW = NC * NS
BPW = B // NW
HALF = BPW // 2
GRP = 16


def _gather6(idx, t_name, t_ds, t_type, t_owner, t_arch, t_task):
    mesh = plsc.VectorSubcoreMesh(core_axis_name="c", subcore_axis_name="s")

    @functools.partial(
        pl.kernel,
        out_type=[jax.ShapeDtypeStruct((B, ED), jnp.float32) for _ in range(6)],
        mesh=mesh,
        scratch_types=[
            pltpu.VMEM((6, BPW + GRP), jnp.int32),
            pltpu.VMEM((HALF, ED), jnp.float32),
            pltpu.VMEM((HALF, ED), jnp.float32),
            pltpu.SemaphoreType.DMA,
            pltpu.SemaphoreType.DMA,
            pltpu.SemaphoreType.DMA,
        ],
    )
    def k(idx_hbm, tn, td, tt, to, ta, tk,
          o0, o1, o2, o3, o4, o5, idx_v, bufa, bufb, gsem, wsa, wsb):
        wid = lax.axis_index("s") * NC + lax.axis_index("c")
        base = wid * BPW
        tabs = [tn, td, tt, to, ta, tk]
        outs = [o0, o1, o2, o3, o4, o5]
        bufs = [bufa, bufb]
        wsems = [wsa, wsb]
        for t in range(6):
            pltpu.sync_copy(idx_hbm.at[t, pl.ds(base, BPW)],
                            idx_v.at[t, pl.ds(0, BPW)])
        pending = [None, None]
        for d in range(12):
            t, h = divmod(d, 2)
            p = d % 2
            if pending[p] is not None:
                pending[p].wait()
            tab = tabs[t]
            buf = bufs[p]

            def grp_body(g, _, t=t, h=h, tab=tab, buf=buf):
                j0 = h * HALF + g * GRP
                iv = idx_v[t, pl.ds(j0, GRP)]
                for kk in range(GRP):
                    pltpu.async_copy(tab.at[pl.ds(iv[kk], 1)],
                                     buf.at[pl.ds(g * GRP + kk, 1)], gsem)
                return 0

            lax.fori_loop(0, HALF // GRP, grp_body, 0)
            pltpu.make_async_copy(tab.at[pl.ds(0, HALF)], buf, gsem).wait()
            pending[p] = pltpu.async_copy(
                buf, outs[t].at[pl.ds(base + h * HALF, HALF)], wsems[p])
        pending[0].wait()
        pending[1].wait()

    return k(idx, t_name, t_ds, t_type, t_owner, t_arch, t_task)


def _mlp_body(e0, e1, e2, e3, e4, e5, nf, w1, b1, w2, b2, out):
    num = jnp.dot(nf[:], w1[:], preferred_element_type=jnp.float32) + b1[:]
    feats = jnp.concatenate([e0[:], e1[:], e2[:], e3[:], e4[:], e5[:], num], axis=-1)
    acc = jnp.dot(feats, w2[:], preferred_element_type=jnp.float32) + b2[:]
    out[:] = jnp.maximum(acc, 0.0)


def _mlp(e_list, nf, w1, b1, w2, b2):
    BB = 2048
    grid = (B // BB,)
    espec = pl.BlockSpec((BB, ED), lambda i: (i, 0))
    return pl.pallas_call(
        _mlp_body,
        grid=grid,
        in_specs=[espec] * 6 + [
            pl.BlockSpec((BB, 6), lambda i: (i, 0)),
            pl.BlockSpec((6, 20), lambda i: (0, 0)),
            pl.BlockSpec((1, 20), lambda i: (0, 0)),
            pl.BlockSpec((116, 64), lambda i: (0, 0)),
            pl.BlockSpec((1, 64), lambda i: (0, 0)),
        ],
        out_specs=pl.BlockSpec((BB, 64), lambda i: (i, 0)),
        out_shape=jax.ShapeDtypeStruct((B, 64), jnp.float32),
        compiler_params=pltpu.CompilerParams(
            dimension_semantics=("parallel",),
        ),
    )(*e_list, nf, w1, b1, w2, b2)


def kernel(model_name, pretrained_dataset, model_type, model_owner,
           model_architecture, model_task, numeric_features,
           T_name, T_ds, T_type, T_owner, T_arch, T_task, W1, b1, W2, b2):
    idx = jnp.stack([
        model_name.astype(jnp.int32),
        pretrained_dataset.astype(jnp.int32),
        model_type.astype(jnp.int32),
        model_owner.astype(jnp.int32),
        model_architecture.astype(jnp.int32),
        model_task.astype(jnp.int32),
    ], axis=0)
    e = _gather6(idx, T_name, T_ds, T_type, T_owner, T_arch, T_task)
    return _mlp(e, numeric_features,
                W1, b1.reshape(1, 20), W2, b2.reshape(1, 64))
